# baseline (device time: 15003 ns/iter reference)
import jax
import jax.numpy as jnp
from jax import lax
from jax.experimental import pallas as pl
from jax.experimental.pallas import tpu as pltpu

N_DEV = 4
B, Sq, Hq, Dh = 2, 128, 4, 64
D_MODEL = 512
DQ = Hq * Dh
BLK = 64


def kernel(x, Wq, K_ext, V_ext, Wo):
    def body(x_ref, wq_ref, k_ref, v_ref, wo_ref, out_ref,
             ctx_ref, send_sems, recv_sem, ack_sem):
        my = lax.axis_index("i")
        barrier = pltpu.get_barrier_semaphore()

        @pl.when(my != 0)
        def _():
            pl.semaphore_signal(
                barrier, inc=1, device_id=(0,),
                device_id_type=pl.DeviceIdType.MESH,
            )

        @pl.when(my == 0)
        def _():
            pl.semaphore_wait(barrier, N_DEV - 1)

            wq = wq_ref[...].astype(jnp.bfloat16)
            ri = lax.broadcasted_iota(jnp.int32, (Sq, Sq), 0) // BLK
            ci = lax.broadcasted_iota(jnp.int32, (Sq, Sq), 1) // BLK
            mask = ci <= ri

            for b in range(B):
                xb = x_ref[b].astype(jnp.bfloat16)
                q = jnp.dot(xb, wq, preferred_element_type=jnp.float32)
                qb16 = q.astype(jnp.bfloat16)
                kb = k_ref[b].reshape(Sq, DQ).astype(jnp.bfloat16)
                vb = v_ref[b].reshape(Sq, DQ).astype(jnp.bfloat16)
                for h in range(Hq):
                    sl = slice(h * Dh, (h + 1) * Dh)
                    scores = lax.dot_general(
                        qb16[:, sl], kb[:, sl],
                        (((1,), (1,)), ((), ())),
                        preferred_element_type=jnp.float32,
                    ) * 0.125
                    scores = jnp.where(mask, scores, -1e9)
                    m = jnp.max(scores, axis=-1, keepdims=True)
                    w = jnp.exp(scores - m)
                    w = w / jnp.sum(w, axis=-1, keepdims=True)
                    ctx_h = jnp.dot(
                        w.astype(jnp.bfloat16), vb[:, sl],
                        preferred_element_type=jnp.float32,
                    )
                    ctx_ref[b, :, sl] = ctx_h.astype(jnp.bfloat16)

            for t in range(1, N_DEV):
                rdma = pltpu.make_async_remote_copy(
                    src_ref=ctx_ref,
                    dst_ref=ctx_ref,
                    send_sem=send_sems.at[t - 1],
                    recv_sem=recv_sem,
                    device_id=(t,),
                    device_id_type=pl.DeviceIdType.MESH,
                )
                rdma.start()
            for t in range(1, N_DEV):
                pltpu.make_async_remote_copy(
                    src_ref=ctx_ref,
                    dst_ref=ctx_ref,
                    send_sem=send_sems.at[t - 1],
                    recv_sem=recv_sem,
                    device_id=(t,),
                    device_id_type=pl.DeviceIdType.MESH,
                ).wait_send()

        @pl.when(my != 0)
        def _():
            pltpu.make_async_remote_copy(
                src_ref=ctx_ref,
                dst_ref=ctx_ref,
                send_sem=send_sems.at[0],
                recv_sem=recv_sem,
                device_id=(0,),
                device_id_type=pl.DeviceIdType.MESH,
            ).wait_recv()
            pl.semaphore_signal(
                ack_sem, inc=1, device_id=(0,),
                device_id_type=pl.DeviceIdType.MESH,
            )

        wo = wo_ref[...].astype(jnp.bfloat16)
        for b in range(B):
            out_ref[b] = jnp.dot(
                ctx_ref[b], wo, preferred_element_type=jnp.float32
            )

        @pl.when(my == 0)
        def _():
            pl.semaphore_wait(ack_sem, N_DEV - 1)

    return pl.pallas_call(
        body,
        out_shape=jax.ShapeDtypeStruct((B, Sq, D_MODEL), jnp.float32),
        in_specs=[pl.BlockSpec(memory_space=pltpu.VMEM)] * 5,
        out_specs=pl.BlockSpec(memory_space=pltpu.VMEM),
        scratch_shapes=[
            pltpu.VMEM((B, Sq, DQ), jnp.bfloat16),
            pltpu.SemaphoreType.DMA((N_DEV - 1,)),
            pltpu.SemaphoreType.DMA,
            pltpu.SemaphoreType.REGULAR,
        ],
        compiler_params=pltpu.CompilerParams(collective_id=0),
    )(x, Wq, K_ext, V_ext, Wo)


# device time: 11320 ns/iter; 1.3254x vs baseline; 1.3254x over previous
import jax
import jax.numpy as jnp
from jax import lax
from jax.experimental import pallas as pl
from jax.experimental.pallas import tpu as pltpu

N_DEV = 4
B, Sq, Hq, Dh = 2, 128, 4, 64
D_MODEL = 512
DQ = Hq * Dh
BLK = 64


def kernel(x, Wq, K_ext, V_ext, Wo):
    def body(x_hbm, wq_hbm, k_hbm, v_hbm, wo_hbm, out_ref,
             xv, wqv, kv, vv, wov, ctx_ref,
             copy_sems, send_sems, recv_sems):
        my = lax.axis_index("i")
        barrier = pltpu.get_barrier_semaphore()

        @pl.when(my != 0)
        def _():
            pl.semaphore_signal(
                barrier, inc=1, device_id=(0,),
                device_id_type=pl.DeviceIdType.MESH,
            )
            cp_wo = pltpu.make_async_copy(wo_hbm, wov, copy_sems.at[4])
            cp_wo.start()
            cp_wo.wait()

        @pl.when(my == 0)
        def _():
            cps = [
                pltpu.make_async_copy(x_hbm, xv, copy_sems.at[0]),
                pltpu.make_async_copy(wq_hbm, wqv, copy_sems.at[1]),
                pltpu.make_async_copy(k_hbm, kv, copy_sems.at[2]),
                pltpu.make_async_copy(v_hbm, vv, copy_sems.at[3]),
                pltpu.make_async_copy(wo_hbm, wov, copy_sems.at[4]),
            ]
            for cp in cps:
                cp.start()
            cps[0].wait()
            cps[1].wait()
            wq16 = (wqv[...] * 0.125).astype(jnp.bfloat16)
            ri = lax.broadcasted_iota(jnp.int32, (Sq, Sq), 0) // BLK
            ci = lax.broadcasted_iota(jnp.int32, (Sq, Sq), 1) // BLK
            mask = ci <= ri

            q16 = []
            for b in range(B):
                xb = xv[b].astype(jnp.bfloat16)
                q16.append(jnp.dot(
                    xb, wq16, preferred_element_type=jnp.float32
                ).astype(jnp.bfloat16))

            cps[2].wait()
            cps[3].wait()
            pl.semaphore_wait(barrier, N_DEV - 1)

            for b in range(B):
                kb = kv[b].reshape(Sq, DQ).astype(jnp.bfloat16)
                vb = vv[b].reshape(Sq, DQ).astype(jnp.bfloat16)
                for h in range(Hq):
                    sl = slice(h * Dh, (h + 1) * Dh)
                    scores = lax.dot_general(
                        q16[b][:, sl], kb[:, sl],
                        (((1,), (1,)), ((), ())),
                        preferred_element_type=jnp.float32,
                    )
                    w = jnp.exp(jnp.where(mask, scores, -30.0))
                    r = 1.0 / jnp.sum(w, axis=-1, keepdims=True)
                    ctx_h = jnp.dot(
                        w.astype(jnp.bfloat16), vb[:, sl],
                        preferred_element_type=jnp.float32,
                    ) * r
                    ctx_ref[b, :, sl] = ctx_h.astype(jnp.bfloat16)
                for j, t in enumerate((2, 1, 3)):
                    pltpu.make_async_remote_copy(
                        src_ref=ctx_ref.at[b],
                        dst_ref=ctx_ref.at[b],
                        send_sem=send_sems.at[j, b],
                        recv_sem=recv_sems.at[b],
                        device_id=(t,),
                        device_id_type=pl.DeviceIdType.MESH,
                    ).start()
            cps[4].wait()

        wo16 = wov[...].astype(jnp.bfloat16)
        for b in range(B):
            @pl.when(my != 0)
            def _():
                pltpu.make_async_remote_copy(
                    src_ref=ctx_ref.at[b],
                    dst_ref=ctx_ref.at[b],
                    send_sem=send_sems.at[0, b],
                    recv_sem=recv_sems.at[b],
                    device_id=(0,),
                    device_id_type=pl.DeviceIdType.MESH,
                ).wait_recv()
            out_ref[b] = jnp.dot(
                ctx_ref[b], wo16, preferred_element_type=jnp.float32
            )

        @pl.when(my == 0)
        def _():
            for j, t in enumerate((2, 1, 3)):
                for b in range(B):
                    pltpu.make_async_remote_copy(
                        src_ref=ctx_ref.at[b],
                        dst_ref=ctx_ref.at[b],
                        send_sem=send_sems.at[j, b],
                        recv_sem=recv_sems.at[b],
                        device_id=(t,),
                        device_id_type=pl.DeviceIdType.MESH,
                    ).wait_send()

    return pl.pallas_call(
        body,
        out_shape=jax.ShapeDtypeStruct((B, Sq, D_MODEL), jnp.float32),
        in_specs=[pl.BlockSpec(memory_space=pltpu.MemorySpace.HBM)] * 5,
        out_specs=pl.BlockSpec(memory_space=pltpu.MemorySpace.VMEM),
        scratch_shapes=[
            pltpu.VMEM((B, Sq, D_MODEL), jnp.float32),
            pltpu.VMEM((D_MODEL, DQ), jnp.float32),
            pltpu.VMEM((B, Sq, Hq, Dh), jnp.float32),
            pltpu.VMEM((B, Sq, Hq, Dh), jnp.float32),
            pltpu.VMEM((DQ, D_MODEL), jnp.float32),
            pltpu.VMEM((B, Sq, DQ), jnp.bfloat16),
            pltpu.SemaphoreType.DMA((5,)),
            pltpu.SemaphoreType.DMA((3, B)),
            pltpu.SemaphoreType.DMA((B,)),
        ],
        compiler_params=pltpu.CompilerParams(collective_id=0),
    )(x, Wq, K_ext, V_ext, Wo)


# device time: 10464 ns/iter; 1.4338x vs baseline; 1.0818x over previous
import jax
import jax.numpy as jnp
from jax import lax
from jax.experimental import pallas as pl
from jax.experimental.pallas import tpu as pltpu

N_DEV = 4
B, Sq, Hq, Dh = 2, 128, 4, 64
D_MODEL = 512
DQ = Hq * Dh
BLK = 64


def kernel(x, Wq, K_ext, V_ext, Wo):
    def body(x_hbm, wq_hbm, k_hbm, v_hbm, wo_hbm, out_ref,
             xv, wqv, kv, vv, wov, ctx_ref,
             copy_sems, send_sems, recv_sems):
        my = lax.axis_index("i")
        barrier = pltpu.get_barrier_semaphore()

        @pl.when(my != 0)
        def _():
            pl.semaphore_signal(
                barrier, inc=1, device_id=(0,),
                device_id_type=pl.DeviceIdType.MESH,
            )
            cp_wo = pltpu.make_async_copy(wo_hbm, wov, copy_sems.at[4])
            cp_wo.start()
            cp_wo.wait()

        @pl.when(my == 0)
        def _():
            cps = [
                pltpu.make_async_copy(x_hbm, xv, copy_sems.at[0]),
                pltpu.make_async_copy(wq_hbm, wqv, copy_sems.at[1]),
                pltpu.make_async_copy(k_hbm, kv, copy_sems.at[2]),
                pltpu.make_async_copy(v_hbm, vv, copy_sems.at[3]),
                pltpu.make_async_copy(wo_hbm, wov, copy_sems.at[4]),
            ]
            for cp in cps:
                cp.start()
            cps[0].wait()
            cps[1].wait()
            wq16 = (wqv[...] * 0.125).astype(jnp.bfloat16)

            q16 = []
            for b in range(B):
                xb = xv[b].astype(jnp.bfloat16)
                q16.append(jnp.dot(
                    xb, wq16, preferred_element_type=jnp.float32
                ).astype(jnp.bfloat16))

            cps[2].wait()
            cps[3].wait()

            for b in range(B):
                kb = kv[b].reshape(Sq, DQ).astype(jnp.bfloat16)
                vb = vv[b].reshape(Sq, DQ).astype(jnp.bfloat16)
                for h in range(Hq):
                    sl = slice(h * Dh, (h + 1) * Dh)
                    qh, kh, vh = q16[b][:, sl], kb[:, sl], vb[:, sl]
                    for lo, hi in ((0, BLK), (BLK, Sq)):
                        s = lax.dot_general(
                            qh[lo:hi], kh[:hi],
                            (((1,), (1,)), ((), ())),
                            preferred_element_type=jnp.float32,
                        )
                        w = jnp.exp(s)
                        r = 1.0 / jnp.sum(w, axis=-1, keepdims=True)
                        ctx_h = jnp.dot(
                            w.astype(jnp.bfloat16), vh[:hi],
                            preferred_element_type=jnp.float32,
                        ) * r
                        ctx_ref[b, lo:hi, sl] = ctx_h.astype(jnp.bfloat16)
                if b == 0:
                    pl.semaphore_wait(barrier, N_DEV - 1)
                for j, t in enumerate((2, 1, 3)):
                    pltpu.make_async_remote_copy(
                        src_ref=ctx_ref.at[b],
                        dst_ref=ctx_ref.at[b],
                        send_sem=send_sems.at[j, b],
                        recv_sem=recv_sems.at[b],
                        device_id=(t,),
                        device_id_type=pl.DeviceIdType.MESH,
                    ).start()
            cps[4].wait()

        wo16 = wov[...].astype(jnp.bfloat16)
        for b in range(B):
            @pl.when(my != 0)
            def _():
                pltpu.make_async_remote_copy(
                    src_ref=ctx_ref.at[b],
                    dst_ref=ctx_ref.at[b],
                    send_sem=send_sems.at[0, b],
                    recv_sem=recv_sems.at[b],
                    device_id=(0,),
                    device_id_type=pl.DeviceIdType.MESH,
                ).wait_recv()
            out_ref[b] = jnp.dot(
                ctx_ref[b], wo16, preferred_element_type=jnp.float32
            )

        @pl.when(my == 0)
        def _():
            for j, t in enumerate((2, 1, 3)):
                for b in range(B):
                    pltpu.make_async_remote_copy(
                        src_ref=ctx_ref.at[b],
                        dst_ref=ctx_ref.at[b],
                        send_sem=send_sems.at[j, b],
                        recv_sem=recv_sems.at[b],
                        device_id=(t,),
                        device_id_type=pl.DeviceIdType.MESH,
                    ).wait_send()

    return pl.pallas_call(
        body,
        out_shape=jax.ShapeDtypeStruct((B, Sq, D_MODEL), jnp.float32),
        in_specs=[pl.BlockSpec(memory_space=pltpu.MemorySpace.HBM)] * 5,
        out_specs=pl.BlockSpec(memory_space=pltpu.MemorySpace.VMEM),
        scratch_shapes=[
            pltpu.VMEM((B, Sq, D_MODEL), jnp.float32),
            pltpu.VMEM((D_MODEL, DQ), jnp.float32),
            pltpu.VMEM((B, Sq, Hq, Dh), jnp.float32),
            pltpu.VMEM((B, Sq, Hq, Dh), jnp.float32),
            pltpu.VMEM((DQ, D_MODEL), jnp.float32),
            pltpu.VMEM((B, Sq, DQ), jnp.bfloat16),
            pltpu.SemaphoreType.DMA((5,)),
            pltpu.SemaphoreType.DMA((3, B)),
            pltpu.SemaphoreType.DMA((B,)),
        ],
        compiler_params=pltpu.CompilerParams(collective_id=0),
    )(x, Wq, K_ext, V_ext, Wo)
